# baseline (speedup 1.0000x reference)
import functools

import jax
import jax.numpy as jnp
from jax import lax
from jax.experimental import pallas as pl
from jax.experimental.pallas import tpu as pltpu
import jax.experimental.pallas.tpu_sc as plsc

N = 10000
E = 320000
D_IN = 128
HID = 16
OUT = 64

NSC = 2
NTILE = 16
NW = NSC * NTILE
NPAD = 10112
ROWS_PT = NPAD // NTILE
EPT = 10240
CB = 128
NB = EPT // CB
ACCW = 32


def _prep_body(x_ref, w1t_ref, b1_ref, hn_ref, nrm_ref):
    h = jnp.dot(x_ref[...], w1t_ref[...], preferred_element_type=jnp.float32)
    h = jnp.maximum(h + b1_ref[...], 0.0)
    nr = jnp.sqrt(jnp.sum(h * h, axis=1, keepdims=True))
    hn_ref[...] = h / (nr + 1e-12)
    nrm_ref[...] = nr


def _prep(x, w1t, b1):
    blk = 1000
    grid = N // blk
    return pl.pallas_call(
        _prep_body,
        grid=(grid,),
        in_specs=[
            pl.BlockSpec((blk, D_IN), lambda i: (i, 0)),
            pl.BlockSpec((D_IN, HID), lambda i: (0, 0)),
            pl.BlockSpec((1, HID), lambda i: (0, 0)),
        ],
        out_specs=[
            pl.BlockSpec((blk, HID), lambda i: (i, 0)),
            pl.BlockSpec((blk, 1), lambda i: (i, 0)),
        ],
        out_shape=[
            jax.ShapeDtypeStruct((N, HID), jnp.float32),
            jax.ShapeDtypeStruct((N, 1), jnp.float32),
        ],
    )(x, w1t, b1)


def _combine_norm_body(a0_ref, a1_ref, hn_ref, nrm_ref):
    s = a0_ref[...] + a1_ref[...]
    h = s[:, 0:HID] / (s[:, HID:HID + 1] + 1e-16)
    nr = jnp.sqrt(jnp.sum(h * h, axis=1, keepdims=True))
    hn_ref[...] = h / (nr + 1e-12)
    nrm_ref[...] = nr


def _combine_norm(a0, a1):
    blk = 1000
    grid = N // blk
    return pl.pallas_call(
        _combine_norm_body,
        grid=(grid,),
        in_specs=[
            pl.BlockSpec((blk, ACCW), lambda i: (i, 0)),
            pl.BlockSpec((blk, ACCW), lambda i: (i, 0)),
        ],
        out_specs=[
            pl.BlockSpec((blk, HID), lambda i: (i, 0)),
            pl.BlockSpec((blk, 1), lambda i: (i, 0)),
        ],
        out_shape=[
            jax.ShapeDtypeStruct((N, HID), jnp.float32),
            jax.ShapeDtypeStruct((N, 1), jnp.float32),
        ],
    )(a0, a1)


def _final_body(a0_ref, a1_ref, w2t_ref, b2_ref, out_ref):
    s = a0_ref[...] + a1_ref[...]
    h = s[:, 0:HID] / (s[:, HID:HID + 1] + 1e-16)
    logits = jnp.dot(h, w2t_ref[...], preferred_element_type=jnp.float32)
    logits = logits + b2_ref[...]
    m = jnp.max(logits, axis=1, keepdims=True)
    lse = m + jnp.log(jnp.sum(jnp.exp(logits - m), axis=1, keepdims=True))
    out_ref[...] = logits - lse


def _final(a0, a1, w2t, b2):
    blk = 1000
    grid = N // blk
    return pl.pallas_call(
        _final_body,
        grid=(grid,),
        in_specs=[
            pl.BlockSpec((blk, ACCW), lambda i: (i, 0)),
            pl.BlockSpec((blk, ACCW), lambda i: (i, 0)),
            pl.BlockSpec((HID, OUT), lambda i: (0, 0)),
            pl.BlockSpec((1, OUT), lambda i: (0, 0)),
        ],
        out_specs=pl.BlockSpec((blk, OUT), lambda i: (i, 0)),
        out_shape=jax.ShapeDtypeStruct((N, OUT), jnp.float32),
    )(a0, a1, w2t, b2)


def _agnn_sc_body(hn_hbm, nrm_hbm, src_hbm, dst_hbm, beta_hbm, zeros_hbm,
                  acc_hbm,
                  nrm_v, src_v, dst_v, beta_v, hsrc0, hsrc1, hdst0, hdst1,
                  contrib0, contrib1, acc_sh, hn_sh, sem0, sem1, ssem0, ssem1):
    hsrc_v = [hsrc0, hsrc1]
    hdst_v = [hdst0, hdst1]
    contrib_v = [contrib0, contrib1]
    sems = [sem0, sem1]
    ssems = [ssem0, ssem1]
    c = lax.axis_index("c")
    s = lax.axis_index("s")
    wid = c * NTILE + s

    pltpu.sync_copy(src_hbm.at[wid], src_v)
    pltpu.sync_copy(dst_hbm.at[wid], dst_v)
    pltpu.sync_copy(nrm_hbm, nrm_v)
    pltpu.sync_copy(beta_hbm, beta_v)

    pltpu.sync_copy(zeros_hbm, acc_sh.at[pl.ds(s * ROWS_PT, ROWS_PT)])
    pltpu.sync_copy(hn_hbm.at[pl.ds(s * ROWS_PT, ROWS_PT)],
                    hn_sh.at[pl.ds(s * ROWS_PT, ROWS_PT)])
    plsc.subcore_barrier()

    beta = beta_v[...]
    absbeta = jnp.abs(beta)
    lane = lax.iota(jnp.int32, 16)

    def issue(j, b):
        pltpu.async_copy(hn_sh.at[src_v.at[j]], hsrc_v[b], sems[b])
        pltpu.async_copy(hn_sh.at[dst_v.at[j]], hdst_v[b], sems[b])

    def compute(j, b):
        @pl.when(j >= 2)
        def _():
            pltpu.make_async_copy(contrib_v[b], acc_sh.at[dst_v.at[j]],
                                  ssems[b]).wait()

        pltpu.make_async_copy(hn_sh.at[src_v.at[j]], hsrc_v[b], sems[b]).wait()
        pltpu.make_async_copy(hn_sh.at[dst_v.at[j]], hdst_v[b], sems[b]).wait()
        for g in range(CB // 16):
            erow = lane + (g * 16)
            a_list = []
            cosv = jnp.zeros((16,), jnp.float32)
            for d in range(HID):
                dd = jnp.full((16,), d, jnp.int32)
                a = plsc.load_gather(hsrc_v[b], [erow, dd])
                bb = plsc.load_gather(hdst_v[b], [erow, dd])
                a_list.append(a)
                cosv = cosv + a * bb
            w = jnp.exp(beta * cosv - absbeta)
            sv = src_v[j, pl.ds(g * 16, 16)]
            nsrc = plsc.load_gather(nrm_v, [sv])
            scale = w * nsrc
            for d in range(HID):
                dd = jnp.full((16,), d, jnp.int32)
                plsc.store_scatter(contrib_v[b], [erow, dd], a_list[d] * scale)
            plsc.store_scatter(contrib_v[b],
                               [erow, jnp.full((16,), HID, jnp.int32)], w)

    issue(0, 0)
    issue(1, 1)

    def batch(i, carry):
        for b in range(2):
            j = 2 * i + b
            compute(j, b)

            @pl.when(j + 2 < NB)
            def _():
                issue(j + 2, b)

            pltpu.async_copy(contrib_v[b], acc_sh.at[dst_v.at[j]], ssems[b],
                             add=True)
        return carry

    lax.fori_loop(0, NB // 2, batch, 0)

    for b in range(2):
        pltpu.make_async_copy(contrib_v[b], acc_sh.at[dst_v.at[NB - 2 + b]],
                              ssems[b]).wait()

    plsc.subcore_barrier()
    pltpu.sync_copy(acc_sh.at[pl.ds(s * ROWS_PT, ROWS_PT)],
                    acc_hbm.at[c, pl.ds(s * ROWS_PT, ROWS_PT)])


@functools.partial(
    pl.kernel,
    out_type=jax.ShapeDtypeStruct((NSC, NPAD, ACCW), jnp.float32),
    mesh=plsc.VectorSubcoreMesh(core_axis_name="c", subcore_axis_name="s"),
    compiler_params=pltpu.CompilerParams(
        needs_layout_passes=False, use_tc_tiling_on_sc=False),
    scratch_types=[
        pltpu.VMEM((NPAD,), jnp.float32),
        pltpu.VMEM((NB, CB), jnp.int32),
        pltpu.VMEM((NB, CB), jnp.int32),
        pltpu.VMEM((16,), jnp.float32),
        pltpu.VMEM((CB, HID), jnp.float32),
        pltpu.VMEM((CB, HID), jnp.float32),
        pltpu.VMEM((CB, HID), jnp.float32),
        pltpu.VMEM((CB, HID), jnp.float32),
        pltpu.VMEM((CB, ACCW), jnp.float32),
        pltpu.VMEM((CB, ACCW), jnp.float32),
        pltpu.VMEM_SHARED((NPAD, ACCW), jnp.float32),
        pltpu.VMEM_SHARED((NPAD, HID), jnp.float32),
        pltpu.SemaphoreType.DMA,
        pltpu.SemaphoreType.DMA,
        pltpu.SemaphoreType.DMA,
        pltpu.SemaphoreType.DMA,
    ],
)
def _agnn_sc(hn_hbm, nrm_hbm, src_hbm, dst_hbm, beta_hbm, zeros_hbm,
             acc_hbm, *scratch):
    _agnn_sc_body(hn_hbm, nrm_hbm, src_hbm, dst_hbm, beta_hbm, zeros_hbm,
                  acc_hbm, *scratch)


def kernel(x, edge_index, W1, b1, W2, b2, beta2):
    src = edge_index[0]
    dst = edge_index[1]
    pad = EPT * NW - E
    padidx = jnp.full((pad,), N, jnp.int32)
    src_g = jnp.concatenate([src, padidx]).reshape(NW, NB, CB)
    dst_g = jnp.concatenate([dst, padidx]).reshape(NW, NB, CB)
    zeros_acc = jnp.zeros((ROWS_PT, ACCW), jnp.float32)

    hn, nrm = _prep(x, W1.T, b1.reshape(1, HID))

    rowpad = jnp.zeros((NPAD - N, HID), jnp.float32)
    npadz = jnp.zeros((NPAD - N,), jnp.float32)

    def layer(hn_, nrm_, betav):
        hn_p = jnp.concatenate([hn_, rowpad], axis=0)
        nrm_p = jnp.concatenate([nrm_.reshape(N), npadz])
        acc = _agnn_sc(hn_p, nrm_p, src_g, dst_g, betav, zeros_acc)
        return acc[0, :N, :], acc[1, :N, :]

    a0, a1 = layer(hn, nrm, jnp.ones((16,), jnp.float32))
    hn1, nrm1 = _combine_norm(a0, a1)
    b0, b1_ = layer(hn1, nrm1, jnp.broadcast_to(beta2, (16,)).astype(jnp.float32))
    return _final(b0, b1_, W2.T, b2.reshape(1, OUT))


# speedup vs baseline: 1.0086x; 1.0086x over previous
import functools

import jax
import jax.numpy as jnp
from jax import lax
from jax.experimental import pallas as pl
from jax.experimental.pallas import tpu as pltpu
import jax.experimental.pallas.tpu_sc as plsc

N = 10000
E = 320000
D_IN = 128
HID = 16
OUT = 64

NSC = 2
NTILE = 16
NW = NSC * NTILE
NPAD = 10112
ROWS_PT = NPAD // NTILE
EPT = 10240
CB = 128
NB = EPT // CB
ACCW = 32


def _prep_body(x_ref, w1t_ref, b1_ref, hn_ref, nrm_ref):
    h = jnp.dot(x_ref[...], w1t_ref[...], preferred_element_type=jnp.float32)
    h = jnp.maximum(h + b1_ref[...], 0.0)
    nr = jnp.sqrt(jnp.sum(h * h, axis=1, keepdims=True))
    hn_ref[...] = h / (nr + 1e-12)
    nrm_ref[...] = nr


def _prep(x, w1t, b1):
    blk = 1000
    grid = N // blk
    return pl.pallas_call(
        _prep_body,
        grid=(grid,),
        in_specs=[
            pl.BlockSpec((blk, D_IN), lambda i: (i, 0)),
            pl.BlockSpec((D_IN, HID), lambda i: (0, 0)),
            pl.BlockSpec((1, HID), lambda i: (0, 0)),
        ],
        out_specs=[
            pl.BlockSpec((blk, HID), lambda i: (i, 0)),
            pl.BlockSpec((blk, 1), lambda i: (i, 0)),
        ],
        out_shape=[
            jax.ShapeDtypeStruct((N, HID), jnp.float32),
            jax.ShapeDtypeStruct((N, 1), jnp.float32),
        ],
    )(x, w1t, b1)


def _combine_norm_body(a0_ref, a1_ref, hn_ref, nrm_ref):
    s = a0_ref[...] + a1_ref[...]
    h = s[:, 0:HID] / (s[:, HID:HID + 1] + 1e-16)
    nr = jnp.sqrt(jnp.sum(h * h, axis=1, keepdims=True))
    hn_ref[...] = h / (nr + 1e-12)
    nrm_ref[...] = nr


def _combine_norm(a0, a1):
    blk = 1000
    grid = N // blk
    return pl.pallas_call(
        _combine_norm_body,
        grid=(grid,),
        in_specs=[
            pl.BlockSpec((blk, ACCW), lambda i: (i, 0)),
            pl.BlockSpec((blk, ACCW), lambda i: (i, 0)),
        ],
        out_specs=[
            pl.BlockSpec((blk, HID), lambda i: (i, 0)),
            pl.BlockSpec((blk, 1), lambda i: (i, 0)),
        ],
        out_shape=[
            jax.ShapeDtypeStruct((N, HID), jnp.float32),
            jax.ShapeDtypeStruct((N, 1), jnp.float32),
        ],
    )(a0, a1)


def _final_body(a0_ref, a1_ref, w2t_ref, b2_ref, out_ref):
    s = a0_ref[...] + a1_ref[...]
    h = s[:, 0:HID] / (s[:, HID:HID + 1] + 1e-16)
    logits = jnp.dot(h, w2t_ref[...], preferred_element_type=jnp.float32)
    logits = logits + b2_ref[...]
    m = jnp.max(logits, axis=1, keepdims=True)
    lse = m + jnp.log(jnp.sum(jnp.exp(logits - m), axis=1, keepdims=True))
    out_ref[...] = logits - lse


def _final(a0, a1, w2t, b2):
    blk = 1000
    grid = N // blk
    return pl.pallas_call(
        _final_body,
        grid=(grid,),
        in_specs=[
            pl.BlockSpec((blk, ACCW), lambda i: (i, 0)),
            pl.BlockSpec((blk, ACCW), lambda i: (i, 0)),
            pl.BlockSpec((HID, OUT), lambda i: (0, 0)),
            pl.BlockSpec((1, OUT), lambda i: (0, 0)),
        ],
        out_specs=pl.BlockSpec((blk, OUT), lambda i: (i, 0)),
        out_shape=jax.ShapeDtypeStruct((N, OUT), jnp.float32),
    )(a0, a1, w2t, b2)


def _agnn_sc_body(hn_hbm, nrm_hbm, src_hbm, dst_hbm, beta_hbm, zeros_hbm,
                  acc_hbm,
                  nrm_v, src_v, dst_v, beta_v, hsrc0, hsrc1, hdst0, hdst1,
                  contrib0, contrib1, acc_sh, hn_sh, sem0, sem1, ssem0, ssem1):
    hsrc_v = [hsrc0, hsrc1]
    hdst_v = [hdst0, hdst1]
    contrib_v = [contrib0, contrib1]
    sems = [sem0, sem1]
    ssems = [ssem0, ssem1]
    c = lax.axis_index("c")
    s = lax.axis_index("s")
    wid = c * NTILE + s

    pltpu.sync_copy(src_hbm.at[wid], src_v)
    pltpu.sync_copy(dst_hbm.at[wid], dst_v)
    pltpu.sync_copy(nrm_hbm, nrm_v)
    pltpu.sync_copy(beta_hbm, beta_v)

    pltpu.sync_copy(zeros_hbm, acc_sh.at[pl.ds(s * ROWS_PT, ROWS_PT)])
    pltpu.sync_copy(hn_hbm.at[pl.ds(s * ROWS_PT, ROWS_PT)],
                    hn_sh.at[pl.ds(s * ROWS_PT, ROWS_PT)])
    plsc.subcore_barrier()

    beta = beta_v[...]
    absbeta = jnp.abs(beta)
    lane = lax.iota(jnp.int32, 16)

    def issue(j, b):
        pltpu.async_copy(hn_sh.at[src_v.at[j]], hsrc_v[b], sems[b])
        pltpu.async_copy(hn_sh.at[dst_v.at[j]], hdst_v[b], sems[b])

    def compute(j, b):
        @pl.when(j >= 2)
        def _():
            pltpu.make_async_copy(contrib_v[b], acc_sh.at[dst_v.at[j]],
                                  ssems[b]).wait()

        pltpu.make_async_copy(hn_sh.at[src_v.at[j]], hsrc_v[b], sems[b]).wait()
        pltpu.make_async_copy(hn_sh.at[dst_v.at[j]], hdst_v[b], sems[b]).wait()
        for g in range(CB // 16):
            erow = lane + (g * 16)
            sv = src_v[j, pl.ds(g * 16, 16)]
            nsrc = plsc.load_gather(nrm_v, [sv])
            a_list = []
            b_list = []
            for d in range(HID):
                dd = jnp.full((16,), d, jnp.int32)
                a_list.append(plsc.load_gather(hsrc_v[b], [erow, dd]))
                b_list.append(plsc.load_gather(hdst_v[b], [erow, dd]))
            parts = []
            for q in range(4):
                p = a_list[4 * q] * b_list[4 * q]
                for d in range(4 * q + 1, 4 * q + 4):
                    p = p + a_list[d] * b_list[d]
                parts.append(p)
            cosv = (parts[0] + parts[1]) + (parts[2] + parts[3])
            w = jnp.exp(beta * cosv - absbeta)
            scale = w * nsrc
            for d in range(HID):
                dd = jnp.full((16,), d, jnp.int32)
                plsc.store_scatter(contrib_v[b], [erow, dd], a_list[d] * scale)
            plsc.store_scatter(contrib_v[b],
                               [erow, jnp.full((16,), HID, jnp.int32)], w)

    issue(0, 0)
    issue(1, 1)

    def batch(i, carry):
        for b in range(2):
            j = 2 * i + b
            compute(j, b)

            @pl.when(j + 2 < NB)
            def _():
                issue(j + 2, b)

            pltpu.async_copy(contrib_v[b], acc_sh.at[dst_v.at[j]], ssems[b],
                             add=True)
        return carry

    lax.fori_loop(0, NB // 2, batch, 0)

    for b in range(2):
        pltpu.make_async_copy(contrib_v[b], acc_sh.at[dst_v.at[NB - 2 + b]],
                              ssems[b]).wait()

    plsc.subcore_barrier()
    pltpu.sync_copy(acc_sh.at[pl.ds(s * ROWS_PT, ROWS_PT)],
                    acc_hbm.at[c, pl.ds(s * ROWS_PT, ROWS_PT)])


@functools.partial(
    pl.kernel,
    out_type=jax.ShapeDtypeStruct((NSC, NPAD, ACCW), jnp.float32),
    mesh=plsc.VectorSubcoreMesh(core_axis_name="c", subcore_axis_name="s"),
    compiler_params=pltpu.CompilerParams(
        needs_layout_passes=False, use_tc_tiling_on_sc=False),
    scratch_types=[
        pltpu.VMEM((NPAD,), jnp.float32),
        pltpu.VMEM((NB, CB), jnp.int32),
        pltpu.VMEM((NB, CB), jnp.int32),
        pltpu.VMEM((16,), jnp.float32),
        pltpu.VMEM((CB, HID), jnp.float32),
        pltpu.VMEM((CB, HID), jnp.float32),
        pltpu.VMEM((CB, HID), jnp.float32),
        pltpu.VMEM((CB, HID), jnp.float32),
        pltpu.VMEM((CB, ACCW), jnp.float32),
        pltpu.VMEM((CB, ACCW), jnp.float32),
        pltpu.VMEM_SHARED((NPAD, ACCW), jnp.float32),
        pltpu.VMEM_SHARED((NPAD, HID), jnp.float32),
        pltpu.SemaphoreType.DMA,
        pltpu.SemaphoreType.DMA,
        pltpu.SemaphoreType.DMA,
        pltpu.SemaphoreType.DMA,
    ],
)
def _agnn_sc(hn_hbm, nrm_hbm, src_hbm, dst_hbm, beta_hbm, zeros_hbm,
             acc_hbm, *scratch):
    _agnn_sc_body(hn_hbm, nrm_hbm, src_hbm, dst_hbm, beta_hbm, zeros_hbm,
                  acc_hbm, *scratch)


def kernel(x, edge_index, W1, b1, W2, b2, beta2):
    src = edge_index[0]
    dst = edge_index[1]
    pad = EPT * NW - E
    padidx = jnp.full((pad,), N, jnp.int32)
    src_g = jnp.concatenate([src, padidx]).reshape(NW, NB, CB)
    dst_g = jnp.concatenate([dst, padidx]).reshape(NW, NB, CB)
    zeros_acc = jnp.zeros((ROWS_PT, ACCW), jnp.float32)

    hn, nrm = _prep(x, W1.T, b1.reshape(1, HID))

    rowpad = jnp.zeros((NPAD - N, HID), jnp.float32)
    npadz = jnp.zeros((NPAD - N,), jnp.float32)

    def layer(hn_, nrm_, betav):
        hn_p = jnp.concatenate([hn_, rowpad], axis=0)
        nrm_p = jnp.concatenate([nrm_.reshape(N), npadz])
        acc = _agnn_sc(hn_p, nrm_p, src_g, dst_g, betav, zeros_acc)
        return acc[0, :N, :], acc[1, :N, :]

    a0, a1 = layer(hn, nrm, jnp.ones((16,), jnp.float32))
    hn1, nrm1 = _combine_norm(a0, a1)
    b0, b1_ = layer(hn1, nrm1, jnp.broadcast_to(beta2, (16,)).astype(jnp.float32))
    return _final(b0, b1_, W2.T, b2.reshape(1, OUT))


# speedup vs baseline: 2.0471x; 2.0297x over previous
import functools

import jax
import jax.numpy as jnp
from jax import lax
from jax.experimental import pallas as pl
from jax.experimental.pallas import tpu as pltpu
import jax.experimental.pallas.tpu_sc as plsc

N = 10000
E = 320000
D_IN = 128
HID = 16
OUT = 64

NSC = 2
NTILE = 16
NW = NSC * NTILE
NPAD = 10112
ROWS_PT = NPAD // NTILE
EPT = 10240
CB = 128
NB = EPT // CB
W17 = HID + 1


def _prep_body(x_ref, w1t_ref, b1_ref, hn_ref):
    h = jnp.dot(x_ref[...], w1t_ref[...], preferred_element_type=jnp.float32)
    h = jnp.maximum(h + b1_ref[...], 0.0)
    nr = jnp.sqrt(jnp.sum(h * h, axis=1, keepdims=True))
    hn_ref[...] = jnp.concatenate([h / (nr + 1e-12), nr], axis=1)


def _prep(x, w1t, b1):
    blk = 1000
    grid = N // blk
    return pl.pallas_call(
        _prep_body,
        grid=(grid,),
        in_specs=[
            pl.BlockSpec((blk, D_IN), lambda i: (i, 0)),
            pl.BlockSpec((D_IN, HID), lambda i: (0, 0)),
            pl.BlockSpec((1, HID), lambda i: (0, 0)),
        ],
        out_specs=pl.BlockSpec((blk, W17), lambda i: (i, 0)),
        out_shape=jax.ShapeDtypeStruct((N, W17), jnp.float32),
    )(x, w1t, b1)


def _combine_norm_body(a0_ref, a1_ref, hn_ref):
    s = a0_ref[...] + a1_ref[...]
    h = s[:, 0:HID] / (s[:, HID:W17] + 1e-16)
    nr = jnp.sqrt(jnp.sum(h * h, axis=1, keepdims=True))
    hn_ref[...] = jnp.concatenate([h / (nr + 1e-12), nr], axis=1)


def _combine_norm(a0, a1):
    blk = 1000
    grid = N // blk
    return pl.pallas_call(
        _combine_norm_body,
        grid=(grid,),
        in_specs=[
            pl.BlockSpec((blk, W17), lambda i: (i, 0)),
            pl.BlockSpec((blk, W17), lambda i: (i, 0)),
        ],
        out_specs=pl.BlockSpec((blk, W17), lambda i: (i, 0)),
        out_shape=jax.ShapeDtypeStruct((N, W17), jnp.float32),
    )(a0, a1)


def _final_body(a0_ref, a1_ref, w2t_ref, b2_ref, out_ref):
    s = a0_ref[...] + a1_ref[...]
    h = s[:, 0:HID] / (s[:, HID:W17] + 1e-16)
    logits = jnp.dot(h, w2t_ref[...], preferred_element_type=jnp.float32)
    logits = logits + b2_ref[...]
    m = jnp.max(logits, axis=1, keepdims=True)
    lse = m + jnp.log(jnp.sum(jnp.exp(logits - m), axis=1, keepdims=True))
    out_ref[...] = logits - lse


def _final(a0, a1, w2t, b2):
    blk = 1000
    grid = N // blk
    return pl.pallas_call(
        _final_body,
        grid=(grid,),
        in_specs=[
            pl.BlockSpec((blk, W17), lambda i: (i, 0)),
            pl.BlockSpec((blk, W17), lambda i: (i, 0)),
            pl.BlockSpec((HID, OUT), lambda i: (0, 0)),
            pl.BlockSpec((1, OUT), lambda i: (0, 0)),
        ],
        out_specs=pl.BlockSpec((blk, OUT), lambda i: (i, 0)),
        out_shape=jax.ShapeDtypeStruct((N, OUT), jnp.float32),
    )(a0, a1, w2t, b2)


def _agnn_sc_body(hn_hbm, src_hbm, dst_hbm, beta_hbm, zeros_hbm,
                  acc_hbm,
                  src_v, dst_v, beta_v, hsrc0, hsrc1, hdst0, hdst1,
                  contrib0, contrib1, acc_sh, hn_sh, sem0, sem1, ssem0, ssem1):
    hsrc_v = [hsrc0, hsrc1]
    hdst_v = [hdst0, hdst1]
    contrib_v = [contrib0, contrib1]
    sems = [sem0, sem1]
    ssems = [ssem0, ssem1]
    c = lax.axis_index("c")
    s = lax.axis_index("s")
    wid = c * NTILE + s

    pltpu.sync_copy(src_hbm.at[wid], src_v)
    pltpu.sync_copy(dst_hbm.at[wid], dst_v)
    pltpu.sync_copy(beta_hbm, beta_v)

    pltpu.sync_copy(zeros_hbm, acc_sh.at[pl.ds(s * ROWS_PT, ROWS_PT)])
    pltpu.sync_copy(hn_hbm.at[pl.ds(s * ROWS_PT, ROWS_PT)],
                    hn_sh.at[pl.ds(s * ROWS_PT, ROWS_PT)])
    plsc.subcore_barrier()

    beta = beta_v[...]
    absbeta = jnp.abs(beta)
    lane = lax.iota(jnp.int32, 16)

    def issue(j, b):
        pltpu.async_copy(hn_sh.at[src_v.at[j]], hsrc_v[b], sems[b])
        pltpu.async_copy(hn_sh.at[dst_v.at[j]], hdst_v[b], sems[b])

    def compute(j, b):
        @pl.when(j >= 2)
        def _():
            pltpu.make_async_copy(contrib_v[b], acc_sh.at[dst_v.at[j]],
                                  ssems[b]).wait()

        pltpu.make_async_copy(hn_sh.at[src_v.at[j]], hsrc_v[b], sems[b]).wait()
        pltpu.make_async_copy(hn_sh.at[dst_v.at[j]], hdst_v[b], sems[b]).wait()
        for g in range(CB // 16):
            erow = lane + (g * 16)
            nsrc = plsc.load_gather(hsrc_v[b], [erow, jnp.full((16,), HID, jnp.int32)])
            a_list = []
            b_list = []
            for d in range(HID):
                dd = jnp.full((16,), d, jnp.int32)
                a_list.append(plsc.load_gather(hsrc_v[b], [erow, dd]))
                b_list.append(plsc.load_gather(hdst_v[b], [erow, dd]))
            parts = []
            for q in range(4):
                p = a_list[4 * q] * b_list[4 * q]
                for d in range(4 * q + 1, 4 * q + 4):
                    p = p + a_list[d] * b_list[d]
                parts.append(p)
            cosv = (parts[0] + parts[1]) + (parts[2] + parts[3])
            w = jnp.exp(beta * cosv - absbeta)
            scale = w * nsrc
            for d in range(HID):
                dd = jnp.full((16,), d, jnp.int32)
                plsc.store_scatter(contrib_v[b], [erow, dd], a_list[d] * scale)
            plsc.store_scatter(contrib_v[b],
                               [erow, jnp.full((16,), HID, jnp.int32)], w)

    issue(0, 0)
    issue(1, 1)

    def batch(i, carry):
        for b in range(2):
            j = 2 * i + b
            compute(j, b)

            @pl.when(j + 2 < NB)
            def _():
                issue(j + 2, b)

            pltpu.async_copy(contrib_v[b], acc_sh.at[dst_v.at[j]], ssems[b],
                             add=True)
        return carry

    lax.fori_loop(0, NB // 2, batch, 0)

    for b in range(2):
        pltpu.make_async_copy(contrib_v[b], acc_sh.at[dst_v.at[NB - 2 + b]],
                              ssems[b]).wait()

    plsc.subcore_barrier()
    pltpu.sync_copy(acc_sh.at[pl.ds(s * ROWS_PT, ROWS_PT)],
                    acc_hbm.at[c, pl.ds(s * ROWS_PT, ROWS_PT)])


@functools.partial(
    pl.kernel,
    out_type=jax.ShapeDtypeStruct((NSC, NPAD, W17), jnp.float32),
    mesh=plsc.VectorSubcoreMesh(core_axis_name="c", subcore_axis_name="s"),
    compiler_params=pltpu.CompilerParams(
        needs_layout_passes=False, use_tc_tiling_on_sc=False),
    scratch_types=[
        pltpu.VMEM((NB, CB), jnp.int32),
        pltpu.VMEM((NB, CB), jnp.int32),
        pltpu.VMEM((16,), jnp.float32),
        pltpu.VMEM((CB, W17), jnp.float32),
        pltpu.VMEM((CB, W17), jnp.float32),
        pltpu.VMEM((CB, W17), jnp.float32),
        pltpu.VMEM((CB, W17), jnp.float32),
        pltpu.VMEM((CB, W17), jnp.float32),
        pltpu.VMEM((CB, W17), jnp.float32),
        pltpu.VMEM_SHARED((NPAD, W17), jnp.float32),
        pltpu.VMEM_SHARED((NPAD, W17), jnp.float32),
        pltpu.SemaphoreType.DMA,
        pltpu.SemaphoreType.DMA,
        pltpu.SemaphoreType.DMA,
        pltpu.SemaphoreType.DMA,
    ],
)
def _agnn_sc(hn_hbm, src_hbm, dst_hbm, beta_hbm, zeros_hbm,
             acc_hbm, *scratch):
    _agnn_sc_body(hn_hbm, src_hbm, dst_hbm, beta_hbm, zeros_hbm,
                  acc_hbm, *scratch)


def kernel(x, edge_index, W1, b1, W2, b2, beta2):
    src = edge_index[0]
    dst = edge_index[1]
    pad = EPT * NW - E
    padidx = jnp.full((pad,), N, jnp.int32)
    src_g = jnp.concatenate([src, padidx]).reshape(NW, NB, CB)
    dst_g = jnp.concatenate([dst, padidx]).reshape(NW, NB, CB)
    zeros_acc = jnp.zeros((ROWS_PT, W17), jnp.float32)
    rowpad = jnp.zeros((NPAD - N, W17), jnp.float32)

    hn = _prep(x, W1.T, b1.reshape(1, HID))

    def layer(hn_, betav):
        hn_p = jnp.concatenate([hn_, rowpad], axis=0)
        acc = _agnn_sc(hn_p, src_g, dst_g, betav, zeros_acc)
        return acc[0, :N, :], acc[1, :N, :]

    a0, a1 = layer(hn, jnp.ones((16,), jnp.float32))
    hn1 = _combine_norm(a0, a1)
    b0, b1_ = layer(hn1, jnp.broadcast_to(beta2, (16,)).astype(jnp.float32))
    return _final(b0, b1_, W2.T, b2.reshape(1, OUT))
